# Initial kernel scaffold; baseline (speedup 1.0000x reference)
import jax
import jax.numpy as jnp
from jax.experimental import pallas as pl


def kernel(x, mask):
    raise NotImplementedError("write your pallas kernel here")


# speedup vs baseline: 5.3284x; 5.3284x over previous
import jax
import jax.numpy as jnp
from jax.experimental import pallas as pl

_TILE_S = 512


def _body(m_ref, x_ref, o_ref):
    m = m_ref[0, 0, :]
    length = jnp.sum(m)
    j = pl.program_id(1)
    pos = jax.lax.broadcasted_iota(jnp.int32, (_TILE_S, 1), 0) + j * _TILE_S
    m_t = m_ref[0, 0, pl.ds(j * _TILE_S, _TILE_S)][:, None]
    keep = jnp.where((pos < length) & (m_t > 0), 1.0, 0.0).astype(x_ref.dtype)
    o_ref[0] = x_ref[0] * keep


def kernel(x, mask):
    B, S, D = x.shape
    m3 = mask.astype(jnp.int32).reshape(B, 1, S)
    grid = (B, S // _TILE_S)
    return pl.pallas_call(
        _body,
        grid=grid,
        in_specs=[
            pl.BlockSpec((1, 1, S), lambda b, j: (b, 0, 0)),
            pl.BlockSpec((1, _TILE_S, D), lambda b, j: (b, j, 0)),
        ],
        out_specs=pl.BlockSpec((1, _TILE_S, D), lambda b, j: (b, j, 0)),
        out_shape=jax.ShapeDtypeStruct((B, S, D), x.dtype),
    )(m3, x)


# speedup vs baseline: 5.9352x; 1.1139x over previous
import jax
import jax.numpy as jnp
from jax.experimental import pallas as pl

_TILE_S = 1024


def _body(m_ref, x_ref, o_ref):
    m = m_ref[0, 0, :]
    length = jnp.sum(m)
    j = pl.program_id(1)
    pos = jax.lax.broadcasted_iota(jnp.int32, (_TILE_S, 1), 0) + j * _TILE_S
    m_t = m_ref[0, 0, pl.ds(j * _TILE_S, _TILE_S)][:, None]
    keep = jnp.where((pos < length) & (m_t > 0), 1.0, 0.0).astype(x_ref.dtype)
    o_ref[0] = x_ref[0] * keep


def kernel(x, mask):
    B, S, D = x.shape
    m3 = mask.astype(jnp.int32).reshape(B, 1, S)
    grid = (B, S // _TILE_S)
    return pl.pallas_call(
        _body,
        grid=grid,
        in_specs=[
            pl.BlockSpec((1, 1, S), lambda b, j: (b, 0, 0)),
            pl.BlockSpec((1, _TILE_S, D), lambda b, j: (b, j, 0)),
        ],
        out_specs=pl.BlockSpec((1, _TILE_S, D), lambda b, j: (b, j, 0)),
        out_shape=jax.ShapeDtypeStruct((B, S, D), x.dtype),
    )(m3, x)


# speedup vs baseline: 6.0805x; 1.0245x over previous
import jax
import jax.numpy as jnp
from jax.experimental import pallas as pl

_TILE_S = 2048


def _body(m_ref, x_ref, o_ref):
    m = m_ref[0, 0, :]
    length = jnp.sum(m)
    j = pl.program_id(1)
    pos = jax.lax.broadcasted_iota(jnp.int32, (_TILE_S, 1), 0) + j * _TILE_S
    m_t = m_ref[0, 0, pl.ds(j * _TILE_S, _TILE_S)][:, None]
    keep = jnp.where((pos < length) & (m_t > 0), 1.0, 0.0).astype(x_ref.dtype)
    o_ref[0] = x_ref[0] * keep


def kernel(x, mask):
    B, S, D = x.shape
    m3 = mask.astype(jnp.int32).reshape(B, 1, S)
    grid = (B, S // _TILE_S)
    return pl.pallas_call(
        _body,
        grid=grid,
        in_specs=[
            pl.BlockSpec((1, 1, S), lambda b, j: (b, 0, 0)),
            pl.BlockSpec((1, _TILE_S, D), lambda b, j: (b, j, 0)),
        ],
        out_specs=pl.BlockSpec((1, _TILE_S, D), lambda b, j: (b, j, 0)),
        out_shape=jax.ShapeDtypeStruct((B, S, D), x.dtype),
    )(m3, x)
